# baseline (device time: 32074 ns/iter reference)
import jax
import jax.numpy as jnp
from jax import lax
from jax.experimental import pallas as pl
from jax.experimental.pallas import tpu as pltpu

N_DEV = 16
N_GRP = 8


def kernel(x, w_mat):
    m_per, k = x.shape
    _, n = w_mat.shape
    n_per = n // N_DEV
    n_grp = 2 * n_per

    def body(x_hbm, w_hbm, out_ref, x_vmem, w_buf, send_buf, recv_buf,
             x_sem, copy_sems, send_sems, recv_sems):
        my = lax.axis_index("i")
        my_g = lax.div(my, 2)


        k_half = k // 2

        def w_dmas(jj, slot):
            g = lax.rem(my_g + 1 + jj, N_GRP)
            return [
                pltpu.make_async_copy(
                    w_hbm.at[pl.ds(h * k_half, k_half),
                             pl.ds(g * n_grp, n_grp)],
                    w_buf.at[slot, pl.ds(h * k_half, k_half)],
                    copy_sems.at[slot, h],
                )
                for h in (0, 1)
            ]

        def w_start(jj, slot):
            for c in w_dmas(jj, slot):
                c.start()

        def w_wait(jj, slot):
            for c in w_dmas(jj, slot):
                c.wait()

        x_dma = pltpu.make_async_copy(x_hbm, x_vmem, x_sem)
        x_dma.start()
        w_start(0, 0)
        w_start(1, 1)
        x_dma.wait()
        x_val = x_vmem[...].astype(jnp.bfloat16)

        def send(t, ss):
            slot_r = lax.rem(t - my + N_DEV, N_DEV) - 1
            rdma = pltpu.make_async_remote_copy(
                src_ref=send_buf.at[ss],
                dst_ref=recv_buf.at[slot_r],
                send_sem=send_sems.at[ss],
                recv_sem=recv_sems.at[slot_r],
                device_id=(t,),
                device_id_type=pl.DeviceIdType.MESH,
            )
            rdma.start()

        w_wait(0, 0)
        w_start(2, 2)
        g0 = lax.rem(my_g + 1, N_GRP)
        blk16_0 = jnp.maximum(
            jnp.dot(x_val, w_buf[0].astype(jnp.bfloat16),
                    preferred_element_type=jnp.float32),
            0.0,
        ).astype(jnp.bfloat16)
        send_buf[0, :, :] = blk16_0[:, :n_per]
        send_buf[1, :, :] = blk16_0[:, n_per:]

        send(2 * g0, 0)
        send(2 * g0 + 1, 1)

        for jj in range(1, N_GRP):
            slot = jj % 3
            w_wait(jj, slot)
            if jj + 2 < N_GRP:
                w_start(jj + 2, (jj + 2) % 3)
            g = lax.rem(my_g + 1 + jj, N_GRP)
            if jj < N_GRP - 1:
                blk16 = jnp.maximum(
                    jnp.dot(x_val, w_buf[slot].astype(jnp.bfloat16),
                            preferred_element_type=jnp.float32),
                    0.0,
                ).astype(jnp.bfloat16)
                send_buf[2 * jj, :, :] = blk16[:, :n_per]
                send(2 * g, 2 * jj)
                send_buf[2 * jj + 1, :, :] = blk16[:, n_per:]
                send(2 * g + 1, 2 * jj + 1)
            else:
                own_half = lax.rem(my, 2)
                partner = my + 1 - 2 * own_half
                par_w = w_buf[slot, :, pl.ds((1 - own_half) * n_per, n_per)]
                blk_par = jnp.maximum(
                    jnp.dot(x_val, par_w.astype(jnp.bfloat16),
                            preferred_element_type=jnp.float32),
                    0.0,
                )
                send_buf[2 * jj, :, :] = blk_par.astype(jnp.bfloat16)
                send(partner, 2 * jj)
                own_w = w_buf[slot, :, pl.ds(own_half * n_per, n_per)]
                out_ref[pl.ds(my * m_per, m_per)] = jnp.maximum(
                    jnp.dot(x_val, own_w.astype(jnp.bfloat16),
                            preferred_element_type=jnp.float32),
                    0.0,
                )

        for sl in list(range(1, 14)) + [0, 14]:
            src = lax.rem(my - (sl + 1) + N_DEV, N_DEV)
            done = pltpu.make_async_remote_copy(
                src_ref=send_buf.at[sl],
                dst_ref=recv_buf.at[sl],
                send_sem=send_sems.at[sl],
                recv_sem=recv_sems.at[sl],
                device_id=(src,),
                device_id_type=pl.DeviceIdType.MESH,
            )
            done.wait_recv()
            out_ref[pl.ds(src * m_per, m_per)] = (
                recv_buf[sl].astype(jnp.float32))
            done.wait_send()

    return pl.pallas_call(
        body,
        out_shape=jax.ShapeDtypeStruct((N_DEV * m_per, n_per), jnp.float32),
        in_specs=[
            pl.BlockSpec(memory_space=pl.ANY),
            pl.BlockSpec(memory_space=pl.ANY),
        ],
        out_specs=pl.BlockSpec(memory_space=pltpu.MemorySpace.VMEM),
        scratch_shapes=[
            pltpu.VMEM((m_per, k), jnp.float32),
            pltpu.VMEM((3, k, n_grp), jnp.float32),
            pltpu.VMEM((N_DEV - 1, m_per, n_per), jnp.bfloat16),
            pltpu.VMEM((N_DEV - 1, m_per, n_per), jnp.bfloat16),
            pltpu.SemaphoreType.DMA,
            pltpu.SemaphoreType.DMA((3, 2)),
            pltpu.SemaphoreType.DMA((N_DEV - 1,)),
            pltpu.SemaphoreType.DMA((N_DEV - 1,)),
        ],
        compiler_params=pltpu.CompilerParams(
            vmem_limit_bytes=60 * 1024 * 1024,
        ),
    )(x, w_mat)


# device time: 25968 ns/iter; 1.2351x vs baseline; 1.2351x over previous
import jax
import jax.numpy as jnp
from jax import lax
from jax.experimental import pallas as pl
from jax.experimental.pallas import tpu as pltpu

N_DEV = 16
N_GRP = 8


def kernel(x, w_mat):
    m_per, k = x.shape
    _, n = w_mat.shape
    n_per = n // N_DEV
    n_grp = 2 * n_per

    def body(x_hbm, w_hbm, out_ref, x_vmem, w_buf, send_buf, recv_buf,
             x_sem, copy_sems, send_sems, recv_sems):
        my = lax.axis_index("i")
        my_g = lax.div(my, 2)

        barrier = pltpu.get_barrier_semaphore()
        for off in range(1, N_DEV):
            pl.semaphore_signal(
                barrier, inc=1,
                device_id=(lax.rem(my + off, N_DEV),),
                device_id_type=pl.DeviceIdType.MESH,
            )

        k_half = k // 2

        def w_dmas(jj, slot):
            g = lax.rem(my_g + 1 + jj, N_GRP)
            return [
                pltpu.make_async_copy(
                    w_hbm.at[pl.ds(h * k_half, k_half),
                             pl.ds(g * n_grp, n_grp)],
                    w_buf.at[slot, pl.ds(h * k_half, k_half)],
                    copy_sems.at[slot, h],
                )
                for h in (0, 1)
            ]

        def w_start(jj, slot):
            for c in w_dmas(jj, slot):
                c.start()

        def w_wait(jj, slot):
            for c in w_dmas(jj, slot):
                c.wait()

        x_dma = pltpu.make_async_copy(x_hbm, x_vmem, x_sem)
        x_dma.start()
        w_start(0, 0)
        w_start(1, 1)
        x_dma.wait()
        x_val = x_vmem[...].astype(jnp.bfloat16)

        def send(t, ss):
            slot_r = lax.rem(t - my + N_DEV, N_DEV) - 1
            rdma = pltpu.make_async_remote_copy(
                src_ref=send_buf.at[ss],
                dst_ref=recv_buf.at[slot_r],
                send_sem=send_sems.at[ss],
                recv_sem=recv_sems.at[slot_r],
                device_id=(t,),
                device_id_type=pl.DeviceIdType.MESH,
            )
            rdma.start()

        w_wait(0, 0)
        w_start(2, 2)
        g0 = lax.rem(my_g + 1, N_GRP)
        blk16_0 = jnp.maximum(
            jnp.dot(x_val, w_buf[0].astype(jnp.bfloat16),
                    preferred_element_type=jnp.float32),
            0.0,
        ).astype(jnp.bfloat16)
        send_buf[0, :, :] = blk16_0[:, :n_per]
        send_buf[1, :, :] = blk16_0[:, n_per:]

        pl.semaphore_wait(barrier, N_DEV - 1)
        send(2 * g0, 0)
        send(2 * g0 + 1, 1)

        for jj in range(1, N_GRP):
            slot = jj % 3
            w_wait(jj, slot)
            if jj + 2 < N_GRP:
                w_start(jj + 2, (jj + 2) % 3)
            g = lax.rem(my_g + 1 + jj, N_GRP)
            if jj < N_GRP - 1:
                blk16 = jnp.maximum(
                    jnp.dot(x_val, w_buf[slot].astype(jnp.bfloat16),
                            preferred_element_type=jnp.float32),
                    0.0,
                ).astype(jnp.bfloat16)
                send_buf[2 * jj, :, :] = blk16[:, :n_per]
                send(2 * g, 2 * jj)
                send_buf[2 * jj + 1, :, :] = blk16[:, n_per:]
                send(2 * g + 1, 2 * jj + 1)
            else:
                own_half = lax.rem(my, 2)
                partner = my + 1 - 2 * own_half
                par_w = w_buf[slot, :, pl.ds((1 - own_half) * n_per, n_per)]
                blk_par = jnp.maximum(
                    jnp.dot(x_val, par_w.astype(jnp.bfloat16),
                            preferred_element_type=jnp.float32),
                    0.0,
                )
                send_buf[2 * jj, :, :] = blk_par.astype(jnp.bfloat16)
                send(partner, 2 * jj)
                own_w = w_buf[slot, :, pl.ds(own_half * n_per, n_per)]
                out_ref[pl.ds(my * m_per, m_per)] = jnp.maximum(
                    jnp.dot(x_val, own_w.astype(jnp.bfloat16),
                            preferred_element_type=jnp.float32),
                    0.0,
                )

        for sl in list(range(1, 14)) + [0, 14]:
            src = lax.rem(my - (sl + 1) + N_DEV, N_DEV)
            done = pltpu.make_async_remote_copy(
                src_ref=send_buf.at[sl],
                dst_ref=recv_buf.at[sl],
                send_sem=send_sems.at[sl],
                recv_sem=recv_sems.at[sl],
                device_id=(src,),
                device_id_type=pl.DeviceIdType.MESH,
            )
            done.wait_recv()
            out_ref[pl.ds(src * m_per, m_per)] = (
                recv_buf[sl].astype(jnp.float32))
            done.wait_send()

    return pl.pallas_call(
        body,
        out_shape=jax.ShapeDtypeStruct((N_DEV * m_per, n_per), jnp.float32),
        in_specs=[
            pl.BlockSpec(memory_space=pl.ANY),
            pl.BlockSpec(memory_space=pl.ANY),
        ],
        out_specs=pl.BlockSpec(memory_space=pltpu.MemorySpace.VMEM),
        scratch_shapes=[
            pltpu.VMEM((m_per, k), jnp.float32),
            pltpu.VMEM((3, k, n_grp), jnp.float32),
            pltpu.VMEM((N_DEV - 1, m_per, n_per), jnp.bfloat16),
            pltpu.VMEM((N_DEV - 1, m_per, n_per), jnp.bfloat16),
            pltpu.SemaphoreType.DMA,
            pltpu.SemaphoreType.DMA((3, 2)),
            pltpu.SemaphoreType.DMA((N_DEV - 1,)),
            pltpu.SemaphoreType.DMA((N_DEV - 1,)),
        ],
        compiler_params=pltpu.CompilerParams(
            collective_id=0,
            vmem_limit_bytes=60 * 1024 * 1024,
        ),
    )(x, w_mat)
